# Initial kernel scaffold; baseline (speedup 1.0000x reference)
import jax
import jax.numpy as jnp
from jax.experimental import pallas as pl


def kernel(h, x, edge_index, edge_attr, v_init, We1, be1, We2, be2, Wc1, bc1, Wc2, Wn1, bn1, Wn2, bn2, Wv1, bv1, Wv2, Wi1, bi1, Wi2, bi2):
    raise NotImplementedError("write your pallas kernel here")


# speedup vs baseline: 3.7718x; 3.7718x over previous
import functools

import jax
import jax.numpy as jnp
from jax import lax
from jax.experimental import pallas as pl
from jax.experimental.pallas import tpu as pltpu
from jax.experimental.pallas import tpu_sc as plsc

_N = 10000
_E = 320000
_D = 128
_H = 64
_TD = 80
_XD = 16

_NC, _NS = 2, 16
_NW = _NC * _NS
_EPW = _E // _NW
_C = 80
_NCHUNK = _EPW // _C
_NBUF = 5
_BN = 1000
_BE = 2000


def _pre_body(h_ref, x_ref, vi_ref, we1a, be1, we1b, wn1a, wv1, bv1, wv2,
              t1_ref, t2_ref, hn_ref, vout_ref):
    hb = h_ref[...]
    xpad = jnp.concatenate(
        [x_ref[...], jnp.zeros((_BN, _XD - 3), jnp.float32)], axis=1)
    a1 = jnp.dot(hb, we1a[...], preferred_element_type=jnp.float32) + be1[...]
    a2 = jnp.dot(hb, we1b[...], preferred_element_type=jnp.float32)
    t1_ref[...] = jnp.concatenate([a1, xpad], axis=1)
    t2_ref[...] = jnp.concatenate([a2, xpad], axis=1)
    hn_ref[...] = jnp.dot(hb, wn1a[...], preferred_element_type=jnp.float32)
    vs = jax.nn.silu(
        jnp.dot(hb, wv1[...], preferred_element_type=jnp.float32) + bv1[...])
    vel = jnp.sum(vs * wv2[...], axis=1, keepdims=True)
    vout_ref[...] = vi_ref[...] * vel


def _tc_pre(h, x, v_init, we1a, be1, we1b, wn1a, wv1, bv1, wv2):
    grid = (_N // _BN,)
    full = lambda r, c: pl.BlockSpec((r, c), lambda i: (0, 0))
    return pl.pallas_call(
        _pre_body,
        grid=grid,
        in_specs=[
            pl.BlockSpec((_BN, _D), lambda i: (i, 0)),
            pl.BlockSpec((_BN, 3), lambda i: (i, 0)),
            pl.BlockSpec((_BN, 3), lambda i: (i, 0)),
            full(_D, _H), full(1, _H), full(_D, _H), full(_D, _H),
            full(_D, _H), full(1, _H), full(1, _H),
        ],
        out_specs=[
            pl.BlockSpec((_BN, _TD), lambda i: (i, 0)),
            pl.BlockSpec((_BN, _TD), lambda i: (i, 0)),
            pl.BlockSpec((_BN, _H), lambda i: (i, 0)),
            pl.BlockSpec((_BN, 3), lambda i: (i, 0)),
        ],
        out_shape=[
            jax.ShapeDtypeStruct((_N, _TD), jnp.float32),
            jax.ShapeDtypeStruct((_N, _TD), jnp.float32),
            jax.ShapeDtypeStruct((_N, _H), jnp.float32),
            jax.ShapeDtypeStruct((_N, 3), jnp.float32),
        ],
    )(h, x, v_init, we1a, be1, we1b, wn1a, wv1, bv1, wv2)


def _sc_gather(row3, col3, t1, t2):
    mesh = plsc.VectorSubcoreMesh(core_axis_name="c", subcore_axis_name="s")
    scratch = (
        [pltpu.VMEM((_NCHUNK, _C), jnp.int32) for _ in range(2)]
        + [pltpu.VMEM((_C, _TD), jnp.float32) for _ in range(2 * _NBUF)]
        + [pltpu.SemaphoreType.DMA for _ in range(2 * _NBUF)]
    )

    @functools.partial(
        pl.kernel,
        out_type=[jax.ShapeDtypeStruct((_E, _TD), jnp.float32),
                  jax.ShapeDtypeStruct((_E, _TD), jnp.float32)],
        mesh=mesh,
        scratch_types=scratch,
        compiler_params=pltpu.CompilerParams(use_tc_tiling_on_sc=False),
    )
    def k(row_h, col_h, t1_h, t2_h, g1_h, g2_h, *s):
        idx_r, idx_c = s[0], s[1]
        bufs1 = s[2:2 + _NBUF]
        bufs2 = s[2 + _NBUF:2 + 2 * _NBUF]
        gsem = s[2 + 2 * _NBUF:2 + 3 * _NBUF]
        wsem = s[2 + 3 * _NBUF:2 + 4 * _NBUF]
        cid = lax.axis_index("c")
        sid = lax.axis_index("s")
        wid = cid * _NS + sid
        base = wid * _EPW
        pltpu.sync_copy(row_h.at[wid], idx_r)
        pltpu.sync_copy(col_h.at[wid], idx_c)

        def group(g, carry):
            gets = []
            for b in range(_NBUF):
                j = g * _NBUF + b
                gets.append(pltpu.async_copy(
                    t1_h.at[idx_r.at[j]], bufs1[b], gsem[b]))
                gets.append(pltpu.async_copy(
                    t2_h.at[idx_c.at[j]], bufs2[b], gsem[b]))
            puts = []
            for b in range(_NBUF):
                j = g * _NBUF + b
                gets[2 * b].wait()
                gets[2 * b + 1].wait()
                dst = base + j * _C
                puts.append(pltpu.async_copy(
                    bufs1[b], g1_h.at[pl.ds(dst, _C)], wsem[b]))
                puts.append(pltpu.async_copy(
                    bufs2[b], g2_h.at[pl.ds(dst, _C)], wsem[b]))
            for p in puts:
                p.wait()
            return carry

        lax.fori_loop(0, _NCHUNK // _NBUF, group, 0)

    return k(row3, col3, t1, t2)


def _edge_body(g1_ref, g2_ref, ea_ref, wd, wa, we2, be2, wi1, bi1, wi2, bi2s,
               wc1, bc1, wc2, mm_ref, xu_ref):
    g1 = g1_ref[...]
    g2 = g2_ref[...]
    a1 = g1[:, :_H]
    a2 = g2[:, :_H]
    d = g1[:, _H:_H + 3] - g2[:, _H:_H + 3]
    dsq = jnp.sum(d * d, axis=1, keepdims=True)
    epre = a1 + a2 + dsq * wd[...] + ea_ref[...] * wa[...]
    m1 = jax.nn.silu(epre)
    m = jax.nn.silu(
        jnp.dot(m1, we2[...], preferred_element_type=jnp.float32) + be2[...])
    i1 = jax.nn.silu(
        jnp.dot(m, wi1[...], preferred_element_type=jnp.float32) + bi1[...])
    e = jax.nn.sigmoid(jnp.sum(i1 * wi2[...], axis=1, keepdims=True)
                       + bi2s[...])
    c1 = jax.nn.silu(
        jnp.dot(m, wc1[...], preferred_element_type=jnp.float32) + bc1[...])
    phi = jnp.sum(c1 * wc2[...], axis=1, keepdims=True)
    mm_ref[...] = e * m
    xu = (e * phi) * d
    xu_ref[...] = jnp.concatenate(
        [xu, jnp.ones((_BE, 1), jnp.float32),
         jnp.zeros((_BE, _XD - 4), jnp.float32)], axis=1)


def _tc_edge(g1, g2, ea, wd, wa, we2, be2, wi1, bi1, wi2, bi2s, wc1, bc1, wc2):
    grid = (_E // _BE,)
    full = lambda r, c: pl.BlockSpec((r, c), lambda i: (0, 0))
    return pl.pallas_call(
        _edge_body,
        grid=grid,
        in_specs=[
            pl.BlockSpec((_BE, _TD), lambda i: (i, 0)),
            pl.BlockSpec((_BE, _TD), lambda i: (i, 0)),
            pl.BlockSpec((_BE, 1), lambda i: (i, 0)),
            full(1, _H), full(1, _H), full(_H, _H), full(1, _H),
            full(_H, _H // 2), full(1, _H // 2), full(1, _H // 2), full(1, 1),
            full(_H, _H), full(1, _H), full(1, _H),
        ],
        out_specs=[
            pl.BlockSpec((_BE, _H), lambda i: (i, 0)),
            pl.BlockSpec((_BE, _XD), lambda i: (i, 0)),
        ],
        out_shape=[
            jax.ShapeDtypeStruct((_E, _H), jnp.float32),
            jax.ShapeDtypeStruct((_E, _XD), jnp.float32),
        ],
    )(g1, g2, ea, wd, wa, we2, be2, wi1, bi1, wi2, bi2s, wc1, bc1, wc2)


def _sc_scatter(row3, mm4, xu4, zm, zx):
    mesh = plsc.VectorSubcoreMesh(core_axis_name="c", subcore_axis_name="s")
    scratch = (
        [pltpu.VMEM((_NCHUNK, _C), jnp.int32)]
        + [pltpu.VMEM((_C, _H), jnp.float32) for _ in range(_NBUF)]
        + [pltpu.VMEM((_C, _XD), jnp.float32) for _ in range(_NBUF)]
        + [pltpu.SemaphoreType.DMA for _ in range(_NBUF)]
        + [pltpu.VMEM_SHARED((_N, _H), jnp.float32),
           pltpu.VMEM_SHARED((_N, _XD), jnp.float32)]
    )

    @functools.partial(
        pl.kernel,
        out_type=[jax.ShapeDtypeStruct((_NC, _N, _H), jnp.float32),
                  jax.ShapeDtypeStruct((_NC, _N, _XD), jnp.float32)],
        mesh=mesh,
        scratch_types=scratch,
        compiler_params=pltpu.CompilerParams(use_tc_tiling_on_sc=False),
    )
    def k(row_h, mm_h, xu_h, zm_h, zx_h, macc_h, xacc_h, *s):
        idx = s[0]
        mbuf = s[1:1 + _NBUF]
        xbuf = s[1 + _NBUF:1 + 2 * _NBUF]
        sem = s[1 + 2 * _NBUF:1 + 3 * _NBUF]
        sh_m = s[1 + 3 * _NBUF]
        sh_x = s[2 + 3 * _NBUF]
        cid = lax.axis_index("c")
        sid = lax.axis_index("s")
        wid = cid * _NS + sid

        @pl.when(sid == 0)
        def _init():
            pltpu.sync_copy(zm_h, sh_m)
            pltpu.sync_copy(zx_h, sh_x)

        plsc.subcore_barrier()
        pltpu.sync_copy(row_h.at[wid], idx)

        def group(g, carry):
            gets = []
            for b in range(_NBUF):
                j = g * _NBUF + b
                gets.append(pltpu.async_copy(mm_h.at[wid, j], mbuf[b], sem[b]))
                gets.append(pltpu.async_copy(xu_h.at[wid, j], xbuf[b], sem[b]))
            for b in range(_NBUF):
                j = g * _NBUF + b
                gets[2 * b].wait()
                gets[2 * b + 1].wait()
                pltpu.sync_copy(mbuf[b], sh_m.at[idx.at[j]], add=True)
                pltpu.sync_copy(xbuf[b], sh_x.at[idx.at[j]], add=True)
            return carry

        lax.fori_loop(0, _NCHUNK // _NBUF, group, 0)
        plsc.subcore_barrier()

        @pl.when(sid == 0)
        def _writeout():
            pltpu.sync_copy(sh_m, macc_h.at[cid])
            pltpu.sync_copy(sh_x, xacc_h.at[cid])

    return k(row3, mm4, xu4, zm, zx)


def _final_body(hn_ref, macc_ref, xacc_ref, x_ref, vout_ref, wn1b, bn1, wn2,
                bn2, hout_ref, xout_ref):
    m_i = macc_ref[0] + macc_ref[1]
    t = jax.nn.silu(
        hn_ref[...]
        + jnp.dot(m_i, wn1b[...], preferred_element_type=jnp.float32)
        + bn1[...])
    hout_ref[...] = jnp.dot(t, wn2[...],
                            preferred_element_type=jnp.float32) + bn2[...]
    agg = xacc_ref[0] + xacc_ref[1]
    deg = agg[:, 3:4]
    aggx = agg[:, 0:3]
    xb = x_ref[...]
    xout_ref[...] = jnp.where(
        deg > 0.0, xb + vout_ref[...] + aggx / jnp.float32(_N - 1), xb)


def _tc_final(hn, macc, xacc, x, v_out, wn1b, bn1, wn2, bn2):
    grid = (_N // _BN,)
    full = lambda r, c: pl.BlockSpec((r, c), lambda i: (0, 0))
    return pl.pallas_call(
        _final_body,
        grid=grid,
        in_specs=[
            pl.BlockSpec((_BN, _H), lambda i: (i, 0)),
            pl.BlockSpec((_NC, _BN, _H), lambda i: (0, i, 0)),
            pl.BlockSpec((_NC, _BN, _XD), lambda i: (0, i, 0)),
            pl.BlockSpec((_BN, 3), lambda i: (i, 0)),
            pl.BlockSpec((_BN, 3), lambda i: (i, 0)),
            full(_H, _H), full(1, _H), full(_H, _D), full(1, _D),
        ],
        out_specs=[
            pl.BlockSpec((_BN, _D), lambda i: (i, 0)),
            pl.BlockSpec((_BN, 3), lambda i: (i, 0)),
        ],
        out_shape=[
            jax.ShapeDtypeStruct((_N, _D), jnp.float32),
            jax.ShapeDtypeStruct((_N, 3), jnp.float32),
        ],
    )(hn, macc, xacc, x, v_out, wn1b, bn1, wn2, bn2)


def kernel(h, x, edge_index, edge_attr, v_init, We1, be1, We2, be2, Wc1, bc1,
           Wc2, Wn1, bn1, Wn2, bn2, Wv1, bv1, Wv2, Wi1, bi1, Wi2, bi2):
    f32 = jnp.float32
    row = edge_index[0]
    col = edge_index[1]

    t1, t2, hn, v_out = _tc_pre(
        h, x, v_init,
        We1[:_D], be1.reshape(1, _H), We1[_D:2 * _D], Wn1[:_D],
        Wv1, bv1.reshape(1, _H), Wv2[:, 0].reshape(1, _H))

    row3 = row.reshape(_NW, _NCHUNK, _C)
    col3 = col.reshape(_NW, _NCHUNK, _C)
    g1, g2 = _sc_gather(row3, col3, t1, t2)

    mm, xu = _tc_edge(
        g1, g2, edge_attr,
        We1[2 * _D].reshape(1, _H), We1[2 * _D + 1].reshape(1, _H),
        We2, be2.reshape(1, _H),
        Wi1, bi1.reshape(1, _H // 2), Wi2[:, 0].reshape(1, _H // 2),
        bi2.reshape(1, 1),
        Wc1, bc1.reshape(1, _H), Wc2[:, 0].reshape(1, _H))

    mm4 = mm.reshape(_NW, _NCHUNK, _C, _H)
    xu4 = xu.reshape(_NW, _NCHUNK, _C, _XD)
    zm = jnp.zeros((_N, _H), f32)
    zx = jnp.zeros((_N, _XD), f32)
    macc, xacc = _sc_scatter(row3, mm4, xu4, zm, zx)

    h_out, x_out = _tc_final(
        hn, macc, xacc, x, v_out,
        Wn1[_D:], bn1.reshape(1, _H), Wn2, bn2.reshape(1, _D))

    return (h_out, x_out, v_out)


# speedup vs baseline: 4.7326x; 1.2547x over previous
import functools

import jax
import jax.numpy as jnp
from jax import lax
from jax.experimental import pallas as pl
from jax.experimental.pallas import tpu as pltpu
from jax.experimental.pallas import tpu_sc as plsc

_N = 10000
_E = 320000
_D = 128
_H = 64
_TD = 80
_XD = 16

_NC, _NS = 2, 16
_NW = _NC * _NS
_EPW = _E // _NW
_C = 80
_NCHUNK = _EPW // _C
_NBUF = 5
_BN = 1000
_BE = 2000


def _pre_body(h_ref, x_ref, vi_ref, we1a, be1, we1b, wn1a, wv1, bv1, wv2,
              t1_ref, t2_ref, hn_ref, vout_ref):
    hb = h_ref[...]
    xpad = jnp.concatenate(
        [x_ref[...], jnp.zeros((_BN, _XD - 3), jnp.float32)], axis=1)
    a1 = jnp.dot(hb, we1a[...], preferred_element_type=jnp.float32) + be1[...]
    a2 = jnp.dot(hb, we1b[...], preferred_element_type=jnp.float32)
    t1_ref[...] = jnp.concatenate([a1, xpad], axis=1)
    t2_ref[...] = jnp.concatenate([a2, xpad], axis=1)
    hn_ref[...] = jnp.dot(hb, wn1a[...], preferred_element_type=jnp.float32)
    vs = jax.nn.silu(
        jnp.dot(hb, wv1[...], preferred_element_type=jnp.float32) + bv1[...])
    vel = jnp.sum(vs * wv2[...], axis=1, keepdims=True)
    vout_ref[...] = vi_ref[...] * vel


def _tc_pre(h, x, v_init, we1a, be1, we1b, wn1a, wv1, bv1, wv2):
    grid = (_N // _BN,)
    full = lambda r, c: pl.BlockSpec((r, c), lambda i: (0, 0))
    return pl.pallas_call(
        _pre_body,
        grid=grid,
        in_specs=[
            pl.BlockSpec((_BN, _D), lambda i: (i, 0)),
            pl.BlockSpec((_BN, 3), lambda i: (i, 0)),
            pl.BlockSpec((_BN, 3), lambda i: (i, 0)),
            full(_D, _H), full(1, _H), full(_D, _H), full(_D, _H),
            full(_D, _H), full(1, _H), full(1, _H),
        ],
        out_specs=[
            pl.BlockSpec((_BN, _TD), lambda i: (i, 0)),
            pl.BlockSpec((_BN, _TD), lambda i: (i, 0)),
            pl.BlockSpec((_BN, _H), lambda i: (i, 0)),
            pl.BlockSpec((_BN, 3), lambda i: (i, 0)),
        ],
        out_shape=[
            jax.ShapeDtypeStruct((_N, _TD), jnp.float32),
            jax.ShapeDtypeStruct((_N, _TD), jnp.float32),
            jax.ShapeDtypeStruct((_N, _H), jnp.float32),
            jax.ShapeDtypeStruct((_N, 3), jnp.float32),
        ],
    )(h, x, v_init, we1a, be1, we1b, wn1a, wv1, bv1, wv2)


def _sc_gather(row, col, t1, t2):
    mesh = plsc.VectorSubcoreMesh(core_axis_name="c", subcore_axis_name="s")
    scratch = (
        [pltpu.VMEM((_EPW,), jnp.int32) for _ in range(2)]
        + [pltpu.VMEM((_C, _TD), jnp.float32) for _ in range(2 * _NBUF)]
        + [pltpu.SemaphoreType.DMA for _ in range(2 * _NBUF)]
    )

    @functools.partial(
        pl.kernel,
        out_type=[jax.ShapeDtypeStruct((_E, _TD), jnp.float32),
                  jax.ShapeDtypeStruct((_E, _TD), jnp.float32)],
        mesh=mesh,
        scratch_types=scratch,
        compiler_params=pltpu.CompilerParams(use_tc_tiling_on_sc=False),
    )
    def k(row_h, col_h, t1_h, t2_h, g1_h, g2_h, *s):
        idx_r, idx_c = s[0], s[1]
        bufs1 = s[2:2 + _NBUF]
        bufs2 = s[2 + _NBUF:2 + 2 * _NBUF]
        gsem = s[2 + 2 * _NBUF:2 + 3 * _NBUF]
        wsem = s[2 + 3 * _NBUF:2 + 4 * _NBUF]
        cid = lax.axis_index("c")
        sid = lax.axis_index("s")
        wid = cid * _NS + sid
        base = wid * _EPW
        pltpu.sync_copy(row_h.at[pl.ds(base, _EPW)], idx_r)
        pltpu.sync_copy(col_h.at[pl.ds(base, _EPW)], idx_c)

        def group(g, carry):
            gets = []
            for b in range(_NBUF):
                j = g * _NBUF + b
                gets.append(pltpu.async_copy(
                    t1_h.at[idx_r.at[pl.ds(j * _C, _C)]], bufs1[b], gsem[b]))
                gets.append(pltpu.async_copy(
                    t2_h.at[idx_c.at[pl.ds(j * _C, _C)]], bufs2[b], gsem[b]))
            puts = []
            for b in range(_NBUF):
                j = g * _NBUF + b
                gets[2 * b].wait()
                gets[2 * b + 1].wait()
                dst = base + j * _C
                puts.append(pltpu.async_copy(
                    bufs1[b], g1_h.at[pl.ds(dst, _C)], wsem[b]))
                puts.append(pltpu.async_copy(
                    bufs2[b], g2_h.at[pl.ds(dst, _C)], wsem[b]))
            for p in puts:
                p.wait()
            return carry

        lax.fori_loop(0, _NCHUNK // _NBUF, group, 0)

    return k(row, col, t1, t2)


def _edge_body(g1_ref, g2_ref, ea_ref, wd, wa, we2, be2, wi1, bi1, wi2, bi2s,
               wc1, bc1, wc2, mx_ref):
    g1 = g1_ref[...]
    g2 = g2_ref[...]
    a1 = g1[:, :_H]
    a2 = g2[:, :_H]
    d = g1[:, _H:_H + 3] - g2[:, _H:_H + 3]
    dsq = jnp.sum(d * d, axis=1, keepdims=True)
    epre = a1 + a2 + dsq * wd[...] + ea_ref[...] * wa[...]
    m1 = jax.nn.silu(epre)
    m = jax.nn.silu(
        jnp.dot(m1, we2[...], preferred_element_type=jnp.float32) + be2[...])
    i1 = jax.nn.silu(
        jnp.dot(m, wi1[...], preferred_element_type=jnp.float32) + bi1[...])
    e = jax.nn.sigmoid(jnp.sum(i1 * wi2[...], axis=1, keepdims=True)
                       + bi2s[...])
    c1 = jax.nn.silu(
        jnp.dot(m, wc1[...], preferred_element_type=jnp.float32) + bc1[...])
    phi = jnp.sum(c1 * wc2[...], axis=1, keepdims=True)
    xu = (e * phi) * d
    mx_ref[...] = jnp.concatenate(
        [e * m, xu, jnp.ones((_BE, 1), jnp.float32),
         jnp.zeros((_BE, _D - _H - 4), jnp.float32)], axis=1)


def _tc_edge(g1, g2, ea, wd, wa, we2, be2, wi1, bi1, wi2, bi2s, wc1, bc1, wc2):
    grid = (_E // _BE,)
    full = lambda r, c: pl.BlockSpec((r, c), lambda i: (0, 0))
    return pl.pallas_call(
        _edge_body,
        grid=grid,
        in_specs=[
            pl.BlockSpec((_BE, _TD), lambda i: (i, 0)),
            pl.BlockSpec((_BE, _TD), lambda i: (i, 0)),
            pl.BlockSpec((_BE, 1), lambda i: (i, 0)),
            full(1, _H), full(1, _H), full(_H, _H), full(1, _H),
            full(_H, _H // 2), full(1, _H // 2), full(1, _H // 2), full(1, 1),
            full(_H, _H), full(1, _H), full(1, _H),
        ],
        out_specs=pl.BlockSpec((_BE, _D), lambda i: (i, 0)),
        out_shape=jax.ShapeDtypeStruct((_E, _D), jnp.float32),
    )(g1, g2, ea, wd, wa, we2, be2, wi1, bi1, wi2, bi2s, wc1, bc1, wc2)


def _sc_scatter(row, mx, zm):
    mesh = plsc.VectorSubcoreMesh(core_axis_name="c", subcore_axis_name="s")
    scratch = (
        [pltpu.VMEM((_C,), jnp.int32) for _ in range(_NBUF)]
        + [pltpu.VMEM((_C, _TD), jnp.float32) for _ in range(_NBUF)]
        + [pltpu.SemaphoreType.DMA for _ in range(_NBUF)]
        + [pltpu.VMEM_SHARED((_N, _TD), jnp.float32)]
    )

    @functools.partial(
        pl.kernel,
        out_type=jax.ShapeDtypeStruct((_NC, _N, _D), jnp.float32),
        mesh=mesh,
        scratch_types=scratch,
        compiler_params=pltpu.CompilerParams(use_tc_tiling_on_sc=False),
    )
    def k(row_h, mx_h, zm_h, macc_h, *s):
        ibuf = s[0:_NBUF]
        mbuf = s[_NBUF:2 * _NBUF]
        sem = s[2 * _NBUF:3 * _NBUF]
        sh_m = s[3 * _NBUF]
        cid = lax.axis_index("c")
        sid = lax.axis_index("s")
        wid = cid * _NS + sid
        base = wid * _EPW

        @pl.when(sid == 0)
        def _init():
            pltpu.sync_copy(zm_h, sh_m)

        plsc.subcore_barrier()

        def group(g, carry):
            gets = []
            for b in range(_NBUF):
                j = g * _NBUF + b
                src = base + j * _C
                gets.append(pltpu.async_copy(
                    row_h.at[pl.ds(src, _C)], ibuf[b], sem[b]))
                gets.append(pltpu.async_copy(
                    mx_h.at[pl.ds(src, _C), pl.ds(0, _TD)], mbuf[b], sem[b]))
            for b in range(_NBUF):
                gets[2 * b].wait()
                gets[2 * b + 1].wait()
                pltpu.sync_copy(mbuf[b], sh_m.at[ibuf[b]], add=True)
            return carry

        lax.fori_loop(0, _NCHUNK // _NBUF, group, 0)
        plsc.subcore_barrier()

        @pl.when(sid == 0)
        def _writeout():
            pltpu.sync_copy(sh_m, macc_h.at[cid, :, pl.ds(0, _TD)])

    return k(row, mx, zm)


def _final_body(hn_ref, macc_ref, x_ref, vout_ref, wn1b, bn1, wn2,
                bn2, hout_ref, xout_ref):
    acc = macc_ref[0] + macc_ref[1]
    m_i = acc[:, :_H]
    t = jax.nn.silu(
        hn_ref[...]
        + jnp.dot(m_i, wn1b[...], preferred_element_type=jnp.float32)
        + bn1[...])
    hout_ref[...] = jnp.dot(t, wn2[...],
                            preferred_element_type=jnp.float32) + bn2[...]
    deg = acc[:, _H + 3:_H + 4]
    aggx = acc[:, _H:_H + 3]
    xb = x_ref[...]
    xout_ref[...] = jnp.where(
        deg > 0.0, xb + vout_ref[...] + aggx / jnp.float32(_N - 1), xb)


def _tc_final(hn, macc, x, v_out, wn1b, bn1, wn2, bn2):
    grid = (_N // _BN,)
    full = lambda r, c: pl.BlockSpec((r, c), lambda i: (0, 0))
    return pl.pallas_call(
        _final_body,
        grid=grid,
        in_specs=[
            pl.BlockSpec((_BN, _H), lambda i: (i, 0)),
            pl.BlockSpec((_NC, _BN, _D), lambda i: (0, i, 0)),
            pl.BlockSpec((_BN, 3), lambda i: (i, 0)),
            pl.BlockSpec((_BN, 3), lambda i: (i, 0)),
            full(_H, _H), full(1, _H), full(_H, _D), full(1, _D),
        ],
        out_specs=[
            pl.BlockSpec((_BN, _D), lambda i: (i, 0)),
            pl.BlockSpec((_BN, 3), lambda i: (i, 0)),
        ],
        out_shape=[
            jax.ShapeDtypeStruct((_N, _D), jnp.float32),
            jax.ShapeDtypeStruct((_N, 3), jnp.float32),
        ],
    )(hn, macc, x, v_out, wn1b, bn1, wn2, bn2)


def kernel(h, x, edge_index, edge_attr, v_init, We1, be1, We2, be2, Wc1, bc1,
           Wc2, Wn1, bn1, Wn2, bn2, Wv1, bv1, Wv2, Wi1, bi1, Wi2, bi2):
    f32 = jnp.float32
    row = edge_index[0]
    col = edge_index[1]

    t1, t2, hn, v_out = _tc_pre(
        h, x, v_init,
        We1[:_D], be1.reshape(1, _H), We1[_D:2 * _D], Wn1[:_D],
        Wv1, bv1.reshape(1, _H), Wv2[:, 0].reshape(1, _H))

    g1, g2 = _sc_gather(row, col, t1, t2)

    mx = _tc_edge(
        g1, g2, edge_attr,
        We1[2 * _D].reshape(1, _H), We1[2 * _D + 1].reshape(1, _H),
        We2, be2.reshape(1, _H),
        Wi1, bi1.reshape(1, _H // 2), Wi2[:, 0].reshape(1, _H // 2),
        bi2.reshape(1, 1),
        Wc1, bc1.reshape(1, _H), Wc2[:, 0].reshape(1, _H))

    zm = jnp.zeros((_N, _TD), f32)
    macc = _sc_scatter(row, mx, zm)

    h_out, x_out = _tc_final(
        hn, macc, x, v_out,
        Wn1[_D:], bn1.reshape(1, _H), Wn2, bn2.reshape(1, _D))

    return (h_out, x_out, v_out)


# speedup vs baseline: 6.3844x; 1.3490x over previous
import functools

import jax
import jax.numpy as jnp
from jax import lax
from jax.experimental import pallas as pl
from jax.experimental.pallas import tpu as pltpu
from jax.experimental.pallas import tpu_sc as plsc

_N = 10000
_E = 320000
_D = 128
_H = 64
_TD = 80
_XD = 16

_NC, _NS = 2, 16
_NW = _NC * _NS
_EPW = _E // _NW
_C = 80
_NCHUNK = _EPW // _C
_NBUF = 5
_BN = 1000
_BE = 2000


def _pre_body(h_ref, x_ref, vi_ref, we1a, be1, we1b, wn1a, wv1, bv1, wv2,
              t1_ref, t2_ref, hn_ref, vout_ref):
    hb = h_ref[...]
    xpad = jnp.concatenate(
        [x_ref[...], jnp.zeros((_BN, _XD - 3), jnp.float32)], axis=1)
    a1 = jnp.dot(hb, we1a[...], preferred_element_type=jnp.float32) + be1[...]
    a2 = jnp.dot(hb, we1b[...], preferred_element_type=jnp.float32)
    t1_ref[...] = jnp.concatenate([a1, xpad], axis=1)
    t2_ref[...] = jnp.concatenate([a2, xpad], axis=1)
    hn_ref[...] = jnp.dot(hb, wn1a[...], preferred_element_type=jnp.float32)
    vs = jax.nn.silu(
        jnp.dot(hb, wv1[...], preferred_element_type=jnp.float32) + bv1[...])
    vel = jnp.sum(vs * wv2[...], axis=1, keepdims=True)
    vout_ref[...] = vi_ref[...] * vel


def _tc_pre(h, x, v_init, we1a, be1, we1b, wn1a, wv1, bv1, wv2):
    grid = (_N // _BN,)
    full = lambda r, c: pl.BlockSpec((r, c), lambda i: (0, 0))
    return pl.pallas_call(
        _pre_body,
        grid=grid,
        in_specs=[
            pl.BlockSpec((_BN, _D), lambda i: (i, 0)),
            pl.BlockSpec((_BN, 3), lambda i: (i, 0)),
            pl.BlockSpec((_BN, 3), lambda i: (i, 0)),
            full(_D, _H), full(1, _H), full(_D, _H), full(_D, _H),
            full(_D, _H), full(1, _H), full(1, _H),
        ],
        out_specs=[
            pl.BlockSpec((_BN, _TD), lambda i: (i, 0)),
            pl.BlockSpec((_BN, _TD), lambda i: (i, 0)),
            pl.BlockSpec((_BN, _H), lambda i: (i, 0)),
            pl.BlockSpec((_BN, 3), lambda i: (i, 0)),
        ],
        out_shape=[
            jax.ShapeDtypeStruct((_N, _TD), jnp.float32),
            jax.ShapeDtypeStruct((_N, _TD), jnp.float32),
            jax.ShapeDtypeStruct((_N, _H), jnp.float32),
            jax.ShapeDtypeStruct((_N, 3), jnp.float32),
        ],
    )(h, x, v_init, we1a, be1, we1b, wn1a, wv1, bv1, wv2)


_GBUF = 4


def _sc_gather(row, col, t1, t2):
    mesh = plsc.VectorSubcoreMesh(core_axis_name="c", subcore_axis_name="s")
    scratch = (
        [pltpu.VMEM((_EPW,), jnp.int32) for _ in range(2)]
        + [pltpu.VMEM((_C, _TD), jnp.float32) for _ in range(2 * _GBUF)]
        + [pltpu.VMEM((_C, _D), jnp.float32) for _ in range(_GBUF)]
        + [pltpu.SemaphoreType.DMA for _ in range(2 * _GBUF)]
    )

    @functools.partial(
        pl.kernel,
        out_type=jax.ShapeDtypeStruct((_E, _D), jnp.float32),
        mesh=mesh,
        scratch_types=scratch,
        compiler_params=pltpu.CompilerParams(use_tc_tiling_on_sc=False),
    )
    def k(row_h, col_h, t1_h, t2_h, mx_h, *s):
        idx_r, idx_c = s[0], s[1]
        bufs1 = s[2:2 + _GBUF]
        bufs2 = s[2 + _GBUF:2 + 2 * _GBUF]
        outb = s[2 + 2 * _GBUF:2 + 3 * _GBUF]
        gsem = s[2 + 3 * _GBUF:2 + 4 * _GBUF]
        wsem = s[2 + 4 * _GBUF:2 + 5 * _GBUF]
        cid = lax.axis_index("c")
        sid = lax.axis_index("s")
        wid = cid * _NS + sid
        base = wid * _EPW
        pltpu.sync_copy(row_h.at[pl.ds(base, _EPW)], idx_r)
        pltpu.sync_copy(col_h.at[pl.ds(base, _EPW)], idx_c)

        def issue_get(j, b):
            ofs = j * _C
            pltpu.async_copy(
                t1_h.at[idx_r.at[pl.ds(ofs, _C)]], bufs1[b], gsem[b])
            pltpu.async_copy(
                t2_h.at[idx_c.at[pl.ds(ofs, _C)]], bufs2[b], gsem[b])

        def wait_get(b):
            pltpu.make_async_copy(t1_h.at[pl.ds(0, _C)], bufs1[b],
                                  gsem[b]).wait()
            pltpu.make_async_copy(t2_h.at[pl.ds(0, _C)], bufs2[b],
                                  gsem[b]).wait()

        def wait_put(b):
            pltpu.make_async_copy(outb[b], mx_h.at[pl.ds(0, _C)],
                                  wsem[b]).wait()

        def compute(j, b):
            def edge(i, carry):
                d16 = (bufs1[b][i, pl.ds(_H, 16)]
                       - bufs2[b][i, pl.ds(_H, 16)])
                for q in range(4):
                    sl = pl.ds(q * 16, 16)
                    outb[b][i, sl] = bufs1[b][i, sl] + bufs2[b][i, sl]
                outb[b][i, pl.ds(_H, 16)] = d16
                return carry

            lax.fori_loop(0, _C, edge, 0)
            pltpu.async_copy(outb[b], mx_h.at[pl.ds(base + j * _C, _C)],
                             wsem[b])

        for b in range(_GBUF):
            issue_get(b, b)

        def group(g, carry):
            for b in range(_GBUF):
                j = g * _GBUF + b
                wait_get(b)

                @pl.when(g > 0)
                def _drain():
                    wait_put(b)

                compute(j, b)

                @pl.when(j + _GBUF < _NCHUNK)
                def _prefetch():
                    issue_get(j + _GBUF, b)

            return carry

        ngroups = _NCHUNK // _GBUF
        lax.fori_loop(0, ngroups, group, 0)
        wait_get(0)
        wait_put(0)
        compute(_NCHUNK - 1, 0)
        for b in range(_GBUF):
            wait_put(b)

    return k(row, col, t1, t2)


def _edge_body(g_ref, ea_ref, wd, wa, we2, be2, wi1, bi1, wi2, bi2s,
               wc1, bc1, wc2, mx_ref):
    g = g_ref[...]
    d = g[:, _H:_H + 3]
    dsq = jnp.sum(d * d, axis=1, keepdims=True)
    epre = g[:, :_H] + dsq * wd[...] + ea_ref[...] * wa[...]
    m1 = jax.nn.silu(epre)
    m = jax.nn.silu(
        jnp.dot(m1, we2[...], preferred_element_type=jnp.float32) + be2[...])
    i1 = jax.nn.silu(
        jnp.dot(m, wi1[...], preferred_element_type=jnp.float32) + bi1[...])
    e = jax.nn.sigmoid(jnp.sum(i1 * wi2[...], axis=1, keepdims=True)
                       + bi2s[...])
    c1 = jax.nn.silu(
        jnp.dot(m, wc1[...], preferred_element_type=jnp.float32) + bc1[...])
    phi = jnp.sum(c1 * wc2[...], axis=1, keepdims=True)
    xu = (e * phi) * d
    mx_ref[...] = jnp.concatenate(
        [e * m, xu, jnp.ones((_BE, 1), jnp.float32),
         jnp.zeros((_BE, _D - _H - 4), jnp.float32)], axis=1)


def _tc_edge(g, ea, wd, wa, we2, be2, wi1, bi1, wi2, bi2s, wc1, bc1, wc2):
    grid = (_E // _BE,)
    full = lambda r, c: pl.BlockSpec((r, c), lambda i: (0, 0))
    return pl.pallas_call(
        _edge_body,
        grid=grid,
        in_specs=[
            pl.BlockSpec((_BE, _D), lambda i: (i, 0)),
            pl.BlockSpec((_BE, 1), lambda i: (i, 0)),
            full(1, _H), full(1, _H), full(_H, _H), full(1, _H),
            full(_H, _H // 2), full(1, _H // 2), full(1, _H // 2), full(1, 1),
            full(_H, _H), full(1, _H), full(1, _H),
        ],
        out_specs=pl.BlockSpec((_BE, _D), lambda i: (i, 0)),
        out_shape=jax.ShapeDtypeStruct((_E, _D), jnp.float32),
    )(g, ea, wd, wa, we2, be2, wi1, bi1, wi2, bi2s, wc1, bc1, wc2)


def _sc_scatter(row, mx, zm):
    mesh = plsc.VectorSubcoreMesh(core_axis_name="c", subcore_axis_name="s")
    scratch = (
        [pltpu.VMEM((_C,), jnp.int32) for _ in range(_NBUF)]
        + [pltpu.VMEM((_C, _TD), jnp.float32) for _ in range(_NBUF)]
        + [pltpu.SemaphoreType.DMA for _ in range(_NBUF)]
        + [pltpu.VMEM_SHARED((_N, _TD), jnp.float32)]
    )

    @functools.partial(
        pl.kernel,
        out_type=jax.ShapeDtypeStruct((_NC, _N, _D), jnp.float32),
        mesh=mesh,
        scratch_types=scratch,
        compiler_params=pltpu.CompilerParams(use_tc_tiling_on_sc=False),
    )
    def k(row_h, mx_h, zm_h, macc_h, *s):
        ibuf = s[0:_NBUF]
        mbuf = s[_NBUF:2 * _NBUF]
        sem = s[2 * _NBUF:3 * _NBUF]
        sh_m = s[3 * _NBUF]
        cid = lax.axis_index("c")
        sid = lax.axis_index("s")
        wid = cid * _NS + sid
        base = wid * _EPW

        @pl.when(sid == 0)
        def _init():
            pltpu.sync_copy(zm_h, sh_m)

        plsc.subcore_barrier()

        def group(g, carry):
            gets = []
            for b in range(_NBUF):
                j = g * _NBUF + b
                src = base + j * _C
                gets.append(pltpu.async_copy(
                    row_h.at[pl.ds(src, _C)], ibuf[b], sem[b]))
                gets.append(pltpu.async_copy(
                    mx_h.at[pl.ds(src, _C), pl.ds(0, _TD)], mbuf[b], sem[b]))
            for b in range(_NBUF):
                gets[2 * b].wait()
                gets[2 * b + 1].wait()
                pltpu.sync_copy(mbuf[b], sh_m.at[ibuf[b]], add=True)
            return carry

        lax.fori_loop(0, _NCHUNK // _NBUF, group, 0)
        plsc.subcore_barrier()

        @pl.when(sid == 0)
        def _writeout():
            pltpu.sync_copy(sh_m, macc_h.at[cid, :, pl.ds(0, _TD)])

    return k(row, mx, zm)


def _final_body(hn_ref, macc_ref, x_ref, vout_ref, wn1b, bn1, wn2,
                bn2, hout_ref, xout_ref):
    acc = macc_ref[0] + macc_ref[1]
    m_i = acc[:, :_H]
    t = jax.nn.silu(
        hn_ref[...]
        + jnp.dot(m_i, wn1b[...], preferred_element_type=jnp.float32)
        + bn1[...])
    hout_ref[...] = jnp.dot(t, wn2[...],
                            preferred_element_type=jnp.float32) + bn2[...]
    deg = acc[:, _H + 3:_H + 4]
    aggx = acc[:, _H:_H + 3]
    xb = x_ref[...]
    xout_ref[...] = jnp.where(
        deg > 0.0, xb + vout_ref[...] + aggx / jnp.float32(_N - 1), xb)


def _tc_final(hn, macc, x, v_out, wn1b, bn1, wn2, bn2):
    grid = (_N // _BN,)
    full = lambda r, c: pl.BlockSpec((r, c), lambda i: (0, 0))
    return pl.pallas_call(
        _final_body,
        grid=grid,
        in_specs=[
            pl.BlockSpec((_BN, _H), lambda i: (i, 0)),
            pl.BlockSpec((_NC, _BN, _D), lambda i: (0, i, 0)),
            pl.BlockSpec((_BN, 3), lambda i: (i, 0)),
            pl.BlockSpec((_BN, 3), lambda i: (i, 0)),
            full(_H, _H), full(1, _H), full(_H, _D), full(1, _D),
        ],
        out_specs=[
            pl.BlockSpec((_BN, _D), lambda i: (i, 0)),
            pl.BlockSpec((_BN, 3), lambda i: (i, 0)),
        ],
        out_shape=[
            jax.ShapeDtypeStruct((_N, _D), jnp.float32),
            jax.ShapeDtypeStruct((_N, 3), jnp.float32),
        ],
    )(hn, macc, x, v_out, wn1b, bn1, wn2, bn2)


def kernel(h, x, edge_index, edge_attr, v_init, We1, be1, We2, be2, Wc1, bc1,
           Wc2, Wn1, bn1, Wn2, bn2, Wv1, bv1, Wv2, Wi1, bi1, Wi2, bi2):
    f32 = jnp.float32
    row = edge_index[0]
    col = edge_index[1]

    t1, t2, hn, v_out = _tc_pre(
        h, x, v_init,
        We1[:_D], be1.reshape(1, _H), We1[_D:2 * _D], Wn1[:_D],
        Wv1, bv1.reshape(1, _H), Wv2[:, 0].reshape(1, _H))

    gx = _sc_gather(row, col, t1, t2)

    mx = _tc_edge(
        gx, edge_attr,
        We1[2 * _D].reshape(1, _H), We1[2 * _D + 1].reshape(1, _H),
        We2, be2.reshape(1, _H),
        Wi1, bi1.reshape(1, _H // 2), Wi2[:, 0].reshape(1, _H // 2),
        bi2.reshape(1, 1),
        Wc1, bc1.reshape(1, _H), Wc2[:, 0].reshape(1, _H))

    zm = jnp.zeros((_N, _TD), f32)
    macc = _sc_scatter(row, mx, zm)

    h_out, x_out = _tc_final(
        hn, macc, x, v_out,
        Wn1[_D:], bn1.reshape(1, _H), Wn2, bn2.reshape(1, _D))

    return (h_out, x_out, v_out)


# speedup vs baseline: 7.5589x; 1.1840x over previous
import functools

import jax
import jax.numpy as jnp
from jax import lax
from jax.experimental import pallas as pl
from jax.experimental.pallas import tpu as pltpu
from jax.experimental.pallas import tpu_sc as plsc

_N = 10000
_E = 320000
_D = 128
_H = 64
_TD = 80
_XD = 16

_NC, _NS = 2, 16
_NW = _NC * _NS
_EPW = _E // _NW
_C = 80
_NCHUNK = _EPW // _C
_NBUF = 5
_BN = 1000
_BE = 2000


def _pre_body(h_ref, x_ref, vi_ref, we1a, be1, we1b, wn1a, wv1, bv1, wv2,
              t1_ref, t2_ref, hn_ref, vout_ref):
    hb = h_ref[...]
    xpad = jnp.concatenate(
        [x_ref[...], jnp.zeros((_BN, _XD - 3), jnp.float32)], axis=1)
    a1 = jnp.dot(hb, we1a[...], preferred_element_type=jnp.float32) + be1[...]
    a2 = jnp.dot(hb, we1b[...], preferred_element_type=jnp.float32)
    t1_ref[...] = jnp.concatenate([a1, xpad], axis=1)
    t2_ref[...] = jnp.concatenate([a2, xpad], axis=1)
    hn_ref[...] = jnp.dot(hb, wn1a[...], preferred_element_type=jnp.float32)
    vs = jax.nn.silu(
        jnp.dot(hb, wv1[...], preferred_element_type=jnp.float32) + bv1[...])
    vel = jnp.sum(vs * wv2[...], axis=1, keepdims=True)
    vout_ref[...] = vi_ref[...] * vel


def _tc_pre(h, x, v_init, we1a, be1, we1b, wn1a, wv1, bv1, wv2):
    grid = (_N // _BN,)
    full = lambda r, c: pl.BlockSpec((r, c), lambda i: (0, 0))
    return pl.pallas_call(
        _pre_body,
        grid=grid,
        in_specs=[
            pl.BlockSpec((_BN, _D), lambda i: (i, 0)),
            pl.BlockSpec((_BN, 3), lambda i: (i, 0)),
            pl.BlockSpec((_BN, 3), lambda i: (i, 0)),
            full(_D, _H), full(1, _H), full(_D, _H), full(_D, _H),
            full(_D, _H), full(1, _H), full(1, _H),
        ],
        out_specs=[
            pl.BlockSpec((_BN, _TD), lambda i: (i, 0)),
            pl.BlockSpec((_BN, _TD), lambda i: (i, 0)),
            pl.BlockSpec((_BN, _H), lambda i: (i, 0)),
            pl.BlockSpec((_BN, 3), lambda i: (i, 0)),
        ],
        out_shape=[
            jax.ShapeDtypeStruct((_N, _TD), jnp.float32),
            jax.ShapeDtypeStruct((_N, _TD), jnp.float32),
            jax.ShapeDtypeStruct((_N, _H), jnp.float32),
            jax.ShapeDtypeStruct((_N, 3), jnp.float32),
        ],
    )(h, x, v_init, we1a, be1, we1b, wn1a, wv1, bv1, wv2)


_GBUF = 4


def _sc_gather(row, col, t1, t2):
    mesh = plsc.VectorSubcoreMesh(core_axis_name="c", subcore_axis_name="s")
    scratch = (
        [pltpu.VMEM((_EPW,), jnp.int32) for _ in range(2)]
        + [pltpu.VMEM((_C, _TD), jnp.float32) for _ in range(2 * _GBUF)]
        + [pltpu.VMEM((_C, _D), jnp.float32) for _ in range(_GBUF)]
        + [pltpu.SemaphoreType.DMA for _ in range(2 * _GBUF)]
    )

    @functools.partial(
        pl.kernel,
        out_type=jax.ShapeDtypeStruct((_E, _D), jnp.float32),
        mesh=mesh,
        scratch_types=scratch,
        compiler_params=pltpu.CompilerParams(use_tc_tiling_on_sc=False),
    )
    def k(row_h, col_h, t1_h, t2_h, mx_h, *s):
        idx_r, idx_c = s[0], s[1]
        bufs1 = s[2:2 + _GBUF]
        bufs2 = s[2 + _GBUF:2 + 2 * _GBUF]
        outb = s[2 + 2 * _GBUF:2 + 3 * _GBUF]
        gsem = s[2 + 3 * _GBUF:2 + 4 * _GBUF]
        wsem = s[2 + 4 * _GBUF:2 + 5 * _GBUF]
        cid = lax.axis_index("c")
        sid = lax.axis_index("s")
        wid = cid * _NS + sid
        base = wid * _EPW
        pltpu.sync_copy(row_h.at[pl.ds(base, _EPW)], idx_r)
        pltpu.sync_copy(col_h.at[pl.ds(base, _EPW)], idx_c)

        def issue_get(j, b):
            ofs = j * _C
            pltpu.async_copy(
                t1_h.at[idx_r.at[pl.ds(ofs, _C)]], bufs1[b], gsem[b])
            pltpu.async_copy(
                t2_h.at[idx_c.at[pl.ds(ofs, _C)]], bufs2[b], gsem[b])

        def wait_get(b):
            pltpu.make_async_copy(t1_h.at[pl.ds(0, _C)], bufs1[b],
                                  gsem[b]).wait()
            pltpu.make_async_copy(t2_h.at[pl.ds(0, _C)], bufs2[b],
                                  gsem[b]).wait()

        def wait_put(b):
            pltpu.make_async_copy(outb[b], mx_h.at[pl.ds(0, _C)],
                                  wsem[b]).wait()

        def compute(j, b):
            def edge(i, carry):
                d16 = (bufs1[b][i, pl.ds(_H, 16)]
                       - bufs2[b][i, pl.ds(_H, 16)])
                for q in range(4):
                    sl = pl.ds(q * 16, 16)
                    outb[b][i, sl] = bufs1[b][i, sl] + bufs2[b][i, sl]
                outb[b][i, pl.ds(_H, 16)] = d16
                return carry

            lax.fori_loop(0, _C, edge, 0)
            pltpu.async_copy(outb[b], mx_h.at[pl.ds(base + j * _C, _C)],
                             wsem[b])

        for b in range(_GBUF):
            issue_get(b, b)

        def group(g, carry):
            for b in range(_GBUF):
                j = g * _GBUF + b
                wait_get(b)

                @pl.when(g > 0)
                def _drain():
                    wait_put(b)

                compute(j, b)

                @pl.when(j + _GBUF < _NCHUNK)
                def _prefetch():
                    issue_get(j + _GBUF, b)

            return carry

        ngroups = _NCHUNK // _GBUF
        lax.fori_loop(0, ngroups, group, 0)
        wait_get(0)
        wait_put(0)
        compute(_NCHUNK - 1, 0)
        for b in range(_GBUF):
            wait_put(b)

    return k(row, col, t1, t2)


def _sigm(x):
    return 0.5 * jnp.tanh(0.5 * x) + 0.5


def _silu(x):
    return x * _sigm(x)


def _edge_body(g_ref, ea_ref, k17, sel3, we2, be2, wi1, bi1, wi2, bi2s,
               wc1, bc1, wc2, mx_ref):
    g = g_ref[...]
    ds = g[:, _H:_H + 16]
    dsea = jnp.concatenate([ds * ds, ea_ref[...]], axis=1)
    epre = (g[:, :_H]
            + jnp.dot(dsea, k17[...], preferred_element_type=jnp.float32))
    m1 = _silu(epre)
    m = _silu(
        jnp.dot(m1, we2[...], preferred_element_type=jnp.float32) + be2[...])
    i1 = _silu(
        jnp.dot(m, wi1[...], preferred_element_type=jnp.float32) + bi1[...])
    e = _sigm(jnp.dot(i1, wi2[...], preferred_element_type=jnp.float32)
              + bi2s[...])
    c1 = _silu(
        jnp.dot(m, wc1[...], preferred_element_type=jnp.float32) + bc1[...])
    phi = jnp.dot(c1, wc2[...], preferred_element_type=jnp.float32)
    xu16 = (e * phi) * ds + sel3[...]
    mx_ref[...] = jnp.concatenate(
        [e * m, xu16, jnp.zeros((_BE, _D - _H - 16), jnp.float32)], axis=1)


def _tc_edge(g, ea, k17, sel3, we2, be2, wi1, bi1, wi2, bi2s, wc1, bc1, wc2):
    grid = (_E // _BE,)
    full = lambda r, c: pl.BlockSpec((r, c), lambda i: (0, 0))
    return pl.pallas_call(
        _edge_body,
        grid=grid,
        in_specs=[
            pl.BlockSpec((_BE, _D), lambda i: (i, 0)),
            pl.BlockSpec((_BE, 1), lambda i: (i, 0)),
            full(17, _H), full(1, 16), full(_H, _H), full(1, _H),
            full(_H, _H // 2), full(1, _H // 2), full(_H // 2, 1), full(1, 1),
            full(_H, _H), full(1, _H), full(_H, 1),
        ],
        out_specs=pl.BlockSpec((_BE, _D), lambda i: (i, 0)),
        out_shape=jax.ShapeDtypeStruct((_E, _D), jnp.float32),
    )(g, ea, k17, sel3, we2, be2, wi1, bi1, wi2, bi2s, wc1, bc1, wc2)


def _sc_scatter(row, mx, zm):
    mesh = plsc.VectorSubcoreMesh(core_axis_name="c", subcore_axis_name="s")
    scratch = (
        [pltpu.VMEM((_C,), jnp.int32) for _ in range(_NBUF)]
        + [pltpu.VMEM((_C, _TD), jnp.float32) for _ in range(_NBUF)]
        + [pltpu.SemaphoreType.DMA for _ in range(_NBUF)]
        + [pltpu.VMEM_SHARED((_N, _TD), jnp.float32)]
    )

    @functools.partial(
        pl.kernel,
        out_type=jax.ShapeDtypeStruct((_NC, _N, _D), jnp.float32),
        mesh=mesh,
        scratch_types=scratch,
        compiler_params=pltpu.CompilerParams(use_tc_tiling_on_sc=False),
    )
    def k(row_h, mx_h, zm_h, macc_h, *s):
        ibuf = s[0:_NBUF]
        mbuf = s[_NBUF:2 * _NBUF]
        sem = s[2 * _NBUF:3 * _NBUF]
        sh_m = s[3 * _NBUF]
        cid = lax.axis_index("c")
        sid = lax.axis_index("s")
        wid = cid * _NS + sid
        base = wid * _EPW

        @pl.when(sid == 0)
        def _init():
            pltpu.sync_copy(zm_h, sh_m)

        plsc.subcore_barrier()

        def group(g, carry):
            gets = []
            for b in range(_NBUF):
                j = g * _NBUF + b
                src = base + j * _C
                gets.append(pltpu.async_copy(
                    row_h.at[pl.ds(src, _C)], ibuf[b], sem[b]))
                gets.append(pltpu.async_copy(
                    mx_h.at[pl.ds(src, _C), pl.ds(0, _TD)], mbuf[b], sem[b]))
            for b in range(_NBUF):
                gets[2 * b].wait()
                gets[2 * b + 1].wait()
                pltpu.sync_copy(mbuf[b], sh_m.at[ibuf[b]], add=True)
            return carry

        lax.fori_loop(0, _NCHUNK // _NBUF, group, 0)
        plsc.subcore_barrier()

        @pl.when(sid == 0)
        def _writeout():
            pltpu.sync_copy(sh_m, macc_h.at[cid, :, pl.ds(0, _TD)])

    return k(row, mx, zm)


def _final_body(hn_ref, macc_ref, x_ref, vout_ref, wn1b, bn1, wn2,
                bn2, hout_ref, xout_ref):
    acc = macc_ref[0] + macc_ref[1]
    m_i = acc[:, :_H]
    t = jax.nn.silu(
        hn_ref[...]
        + jnp.dot(m_i, wn1b[...], preferred_element_type=jnp.float32)
        + bn1[...])
    hout_ref[...] = jnp.dot(t, wn2[...],
                            preferred_element_type=jnp.float32) + bn2[...]
    deg = acc[:, _H + 3:_H + 4]
    aggx = acc[:, _H:_H + 3]
    xb = x_ref[...]
    xout_ref[...] = jnp.where(
        deg > 0.0, xb + vout_ref[...] + aggx / jnp.float32(_N - 1), xb)


def _tc_final(hn, macc, x, v_out, wn1b, bn1, wn2, bn2):
    grid = (_N // _BN,)
    full = lambda r, c: pl.BlockSpec((r, c), lambda i: (0, 0))
    return pl.pallas_call(
        _final_body,
        grid=grid,
        in_specs=[
            pl.BlockSpec((_BN, _H), lambda i: (i, 0)),
            pl.BlockSpec((_NC, _BN, _D), lambda i: (0, i, 0)),
            pl.BlockSpec((_BN, 3), lambda i: (i, 0)),
            pl.BlockSpec((_BN, 3), lambda i: (i, 0)),
            full(_H, _H), full(1, _H), full(_H, _D), full(1, _D),
        ],
        out_specs=[
            pl.BlockSpec((_BN, _D), lambda i: (i, 0)),
            pl.BlockSpec((_BN, 3), lambda i: (i, 0)),
        ],
        out_shape=[
            jax.ShapeDtypeStruct((_N, _D), jnp.float32),
            jax.ShapeDtypeStruct((_N, 3), jnp.float32),
        ],
    )(hn, macc, x, v_out, wn1b, bn1, wn2, bn2)


def kernel(h, x, edge_index, edge_attr, v_init, We1, be1, We2, be2, Wc1, bc1,
           Wc2, Wn1, bn1, Wn2, bn2, Wv1, bv1, Wv2, Wi1, bi1, Wi2, bi2):
    f32 = jnp.float32
    row = edge_index[0]
    col = edge_index[1]

    t1, t2, hn, v_out = _tc_pre(
        h, x, v_init,
        We1[:_D], be1.reshape(1, _H), We1[_D:2 * _D], Wn1[:_D],
        Wv1, bv1.reshape(1, _H), Wv2[:, 0].reshape(1, _H))

    gx = _sc_gather(row, col, t1, t2)

    k17 = jnp.concatenate(
        [jnp.tile(We1[2 * _D].reshape(1, _H), (3, 1)),
         jnp.zeros((13, _H), f32),
         We1[2 * _D + 1].reshape(1, _H)], axis=0)
    sel3 = jnp.zeros((1, 16), f32).at[0, 3].set(1.0)
    mx = _tc_edge(
        gx, edge_attr,
        k17, sel3,
        We2, be2.reshape(1, _H),
        Wi1, bi1.reshape(1, _H // 2), Wi2,
        bi2.reshape(1, 1),
        Wc1, bc1.reshape(1, _H), Wc2)

    zm = jnp.zeros((_N, _TD), f32)
    macc = _sc_scatter(row, mx, zm)

    h_out, x_out = _tc_final(
        hn, macc, x, v_out,
        Wn1[_D:], bn1.reshape(1, _H), Wn2, bn2.reshape(1, _D))

    return (h_out, x_out, v_out)


# speedup vs baseline: 8.5316x; 1.1287x over previous
import functools

import jax
import jax.numpy as jnp
from jax import lax
from jax.experimental import pallas as pl
from jax.experimental.pallas import tpu as pltpu
from jax.experimental.pallas import tpu_sc as plsc

_N = 10000
_E = 320000
_D = 128
_H = 64
_TD = 80
_XD = 16

_NC, _NS = 2, 16
_NW = _NC * _NS
_P = 2
_ES = _E // _P
_EPW = _ES // _NW
_C = 40
_NCHUNK = _EPW // _C
_NBUF = 5
_BN = 1000
_BE = 2000


def _pre_body(h_ref, x_ref, vi_ref, we1a, be1, we1b, wn1a, wv1, bv1, wv2,
              t1_ref, t2_ref, hn_ref, vout_ref):
    hb = h_ref[...]
    xpad = jnp.concatenate(
        [x_ref[...], jnp.zeros((_BN, _XD - 3), jnp.float32)], axis=1)
    a1 = jnp.dot(hb, we1a[...], preferred_element_type=jnp.float32) + be1[...]
    a2 = jnp.dot(hb, we1b[...], preferred_element_type=jnp.float32)
    t1_ref[...] = jnp.concatenate([a1, xpad], axis=1)
    t2_ref[...] = jnp.concatenate([a2, xpad], axis=1)
    hn_ref[...] = jnp.dot(hb, wn1a[...], preferred_element_type=jnp.float32)
    vs = jax.nn.silu(
        jnp.dot(hb, wv1[...], preferred_element_type=jnp.float32) + bv1[...])
    vel = jnp.sum(vs * wv2[...], axis=1, keepdims=True)
    vout_ref[...] = vi_ref[...] * vel


def _tc_pre(h, x, v_init, we1a, be1, we1b, wn1a, wv1, bv1, wv2):
    grid = (_N // _BN,)
    full = lambda r, c: pl.BlockSpec((r, c), lambda i: (0, 0))
    return pl.pallas_call(
        _pre_body,
        grid=grid,
        in_specs=[
            pl.BlockSpec((_BN, _D), lambda i: (i, 0)),
            pl.BlockSpec((_BN, 3), lambda i: (i, 0)),
            pl.BlockSpec((_BN, 3), lambda i: (i, 0)),
            full(_D, _H), full(1, _H), full(_D, _H), full(_D, _H),
            full(_D, _H), full(1, _H), full(1, _H),
        ],
        out_specs=[
            pl.BlockSpec((_BN, _TD), lambda i: (i, 0)),
            pl.BlockSpec((_BN, _TD), lambda i: (i, 0)),
            pl.BlockSpec((_BN, _H), lambda i: (i, 0)),
            pl.BlockSpec((_BN, 3), lambda i: (i, 0)),
        ],
        out_shape=[
            jax.ShapeDtypeStruct((_N, _TD), jnp.float32),
            jax.ShapeDtypeStruct((_N, _TD), jnp.float32),
            jax.ShapeDtypeStruct((_N, _H), jnp.float32),
            jax.ShapeDtypeStruct((_N, 3), jnp.float32),
        ],
    )(h, x, v_init, we1a, be1, we1b, wn1a, wv1, bv1, wv2)


_GBUF = 5


def _sc_gather(row, col, t1, t2, ebase):
    mesh = plsc.VectorSubcoreMesh(core_axis_name="c", subcore_axis_name="s")
    scratch = (
        [pltpu.VMEM((_EPW,), jnp.int32) for _ in range(2)]
        + [pltpu.VMEM((_C, _TD), jnp.float32) for _ in range(2 * _GBUF)]
        + [pltpu.VMEM((_C, _D), jnp.float32) for _ in range(_GBUF)]
        + [pltpu.SemaphoreType.DMA for _ in range(2 * _GBUF)]
    )

    @functools.partial(
        pl.kernel,
        out_type=jax.ShapeDtypeStruct((_ES, _D), jnp.float32),
        mesh=mesh,
        scratch_types=scratch,
        compiler_params=pltpu.CompilerParams(use_tc_tiling_on_sc=False),
    )
    def k(row_h, col_h, t1_h, t2_h, mx_h, *s):
        idx_r, idx_c = s[0], s[1]
        bufs1 = s[2:2 + _GBUF]
        bufs2 = s[2 + _GBUF:2 + 2 * _GBUF]
        outb = s[2 + 2 * _GBUF:2 + 3 * _GBUF]
        gsem = s[2 + 3 * _GBUF:2 + 4 * _GBUF]
        wsem = s[2 + 4 * _GBUF:2 + 5 * _GBUF]
        cid = lax.axis_index("c")
        sid = lax.axis_index("s")
        wid = cid * _NS + sid
        base = wid * _EPW
        pltpu.sync_copy(row_h.at[pl.ds(ebase + base, _EPW)], idx_r)
        pltpu.sync_copy(col_h.at[pl.ds(ebase + base, _EPW)], idx_c)

        def issue_get(j, b):
            ofs = j * _C
            pltpu.async_copy(
                t1_h.at[idx_r.at[pl.ds(ofs, _C)]], bufs1[b], gsem[b])
            pltpu.async_copy(
                t2_h.at[idx_c.at[pl.ds(ofs, _C)]], bufs2[b], gsem[b])

        def wait_get(b):
            pltpu.make_async_copy(t1_h.at[pl.ds(0, _C)], bufs1[b],
                                  gsem[b]).wait()
            pltpu.make_async_copy(t2_h.at[pl.ds(0, _C)], bufs2[b],
                                  gsem[b]).wait()

        def wait_put(b):
            pltpu.make_async_copy(outb[b], mx_h.at[pl.ds(0, _C)],
                                  wsem[b]).wait()

        def compute(j, b):
            def edge(i, carry):
                d16 = (bufs1[b][i, pl.ds(_H, 16)]
                       - bufs2[b][i, pl.ds(_H, 16)])
                for q in range(4):
                    sl = pl.ds(q * 16, 16)
                    outb[b][i, sl] = bufs1[b][i, sl] + bufs2[b][i, sl]
                outb[b][i, pl.ds(_H, 16)] = d16
                return carry

            lax.fori_loop(0, _C, edge, 0)
            pltpu.async_copy(outb[b], mx_h.at[pl.ds(base + j * _C, _C)],
                             wsem[b])

        for b in range(_GBUF):
            issue_get(b, b)

        def group(g, carry):
            for b in range(_GBUF):
                j = g * _GBUF + b
                wait_get(b)

                @pl.when(g > 0)
                def _drain():
                    wait_put(b)

                compute(j, b)

                @pl.when(j + _GBUF < _NCHUNK)
                def _prefetch():
                    issue_get(j + _GBUF, b)

            return carry

        lax.fori_loop(0, _NCHUNK // _GBUF, group, 0)
        for b in range(_GBUF):
            wait_put(b)

    return k(row, col, t1, t2)


def _sigm(x):
    return 0.5 * jnp.tanh(0.5 * x) + 0.5


def _silu(x):
    return x * _sigm(x)


def _edge_body(g_ref, ea_ref, k17, sel3, we2, be2, wi1, bi1, wi2, bi2s,
               wc1, bc1, wc2, mx_ref):
    g = g_ref[...]
    ds = g[:, _H:_H + 16]
    dsea = jnp.concatenate([ds * ds, ea_ref[...]], axis=1)
    epre = (g[:, :_H]
            + jnp.dot(dsea, k17[...], preferred_element_type=jnp.float32))
    m1 = _silu(epre)
    m = _silu(
        jnp.dot(m1, we2[...], preferred_element_type=jnp.float32) + be2[...])
    i1 = _silu(
        jnp.dot(m, wi1[...], preferred_element_type=jnp.float32) + bi1[...])
    e = _sigm(jnp.dot(i1, wi2[...], preferred_element_type=jnp.float32)
              + bi2s[...])
    c1 = _silu(
        jnp.dot(m, wc1[...], preferred_element_type=jnp.float32) + bc1[...])
    phi = jnp.dot(c1, wc2[...], preferred_element_type=jnp.float32)
    xu16 = (e * phi) * ds + sel3[...]
    mx_ref[...] = jnp.concatenate(
        [e * m, xu16, jnp.zeros((_BE, _D - _H - 16), jnp.float32)], axis=1)


def _tc_edge(g, ea, k17, sel3, we2, be2, wi1, bi1, wi2, bi2s, wc1, bc1, wc2,
             blk0):
    grid = (_ES // _BE,)
    full = lambda r, c: pl.BlockSpec((r, c), lambda i: (0, 0))
    return pl.pallas_call(
        _edge_body,
        grid=grid,
        in_specs=[
            pl.BlockSpec((_BE, _D), lambda i: (i, 0)),
            pl.BlockSpec((_BE, 1), lambda i: (i + blk0, 0)),
            full(17, _H), full(1, 16), full(_H, _H), full(1, _H),
            full(_H, _H // 2), full(1, _H // 2), full(_H // 2, 1), full(1, 1),
            full(_H, _H), full(1, _H), full(_H, 1),
        ],
        out_specs=pl.BlockSpec((_BE, _D), lambda i: (i, 0)),
        out_shape=jax.ShapeDtypeStruct((_ES, _D), jnp.float32),
    )(g, ea, k17, sel3, we2, be2, wi1, bi1, wi2, bi2s, wc1, bc1, wc2)


def _sc_scatter(row, mx, zm, ebase):
    mesh = plsc.VectorSubcoreMesh(core_axis_name="c", subcore_axis_name="s")
    scratch = (
        [pltpu.VMEM((_C,), jnp.int32) for _ in range(_NBUF)]
        + [pltpu.VMEM((_C, _TD), jnp.float32) for _ in range(_NBUF)]
        + [pltpu.SemaphoreType.DMA for _ in range(_NBUF)]
        + [pltpu.VMEM_SHARED((_N, _TD), jnp.float32)]
    )

    @functools.partial(
        pl.kernel,
        out_type=jax.ShapeDtypeStruct((_NC, _N, _D), jnp.float32),
        mesh=mesh,
        scratch_types=scratch,
        compiler_params=pltpu.CompilerParams(use_tc_tiling_on_sc=False),
    )
    def k(row_h, mx_h, zm_h, macc_h, *s):
        ibuf = s[0:_NBUF]
        mbuf = s[_NBUF:2 * _NBUF]
        sem = s[2 * _NBUF:3 * _NBUF]
        sh_m = s[3 * _NBUF]
        cid = lax.axis_index("c")
        sid = lax.axis_index("s")
        wid = cid * _NS + sid
        base = wid * _EPW

        @pl.when(sid == 0)
        def _init():
            pltpu.sync_copy(zm_h, sh_m)

        plsc.subcore_barrier()

        def group(g, carry):
            gets = []
            for b in range(_NBUF):
                j = g * _NBUF + b
                src = base + j * _C
                gets.append(pltpu.async_copy(
                    row_h.at[pl.ds(ebase + src, _C)], ibuf[b], sem[b]))
                gets.append(pltpu.async_copy(
                    mx_h.at[pl.ds(src, _C), pl.ds(0, _TD)], mbuf[b], sem[b]))
            for b in range(_NBUF):
                gets[2 * b].wait()
                gets[2 * b + 1].wait()
                pltpu.sync_copy(mbuf[b], sh_m.at[ibuf[b]], add=True)
            return carry

        lax.fori_loop(0, _NCHUNK // _NBUF, group, 0)
        plsc.subcore_barrier()

        @pl.when(sid == 0)
        def _writeout():
            pltpu.sync_copy(sh_m, macc_h.at[cid, :, pl.ds(0, _TD)])

    return k(row, mx, zm)


def _final_body(hn_ref, *refs):
    (macc_refs, (x_ref, vout_ref, wn1b, bn1, wn2, bn2,
                 hout_ref, xout_ref)) = refs[:_P], refs[_P:]
    acc = macc_refs[0][0] + macc_refs[0][1]
    for r in macc_refs[1:]:
        acc = acc + r[0] + r[1]
    m_i = acc[:, :_H]
    t = jax.nn.silu(
        hn_ref[...]
        + jnp.dot(m_i, wn1b[...], preferred_element_type=jnp.float32)
        + bn1[...])
    hout_ref[...] = jnp.dot(t, wn2[...],
                            preferred_element_type=jnp.float32) + bn2[...]
    deg = acc[:, _H + 3:_H + 4]
    aggx = acc[:, _H:_H + 3]
    xb = x_ref[...]
    xout_ref[...] = jnp.where(
        deg > 0.0, xb + vout_ref[...] + aggx / jnp.float32(_N - 1), xb)


def _tc_final(hn, maccs, x, v_out, wn1b, bn1, wn2, bn2):
    grid = (_N // _BN,)
    full = lambda r, c: pl.BlockSpec((r, c), lambda i: (0, 0))
    return pl.pallas_call(
        _final_body,
        grid=grid,
        in_specs=[
            pl.BlockSpec((_BN, _H), lambda i: (i, 0)),
        ] + [
            pl.BlockSpec((_NC, _BN, _D), lambda i: (0, i, 0))
            for _ in range(_P)
        ] + [
            pl.BlockSpec((_BN, 3), lambda i: (i, 0)),
            pl.BlockSpec((_BN, 3), lambda i: (i, 0)),
            full(_H, _H), full(1, _H), full(_H, _D), full(1, _D),
        ],
        out_specs=[
            pl.BlockSpec((_BN, _D), lambda i: (i, 0)),
            pl.BlockSpec((_BN, 3), lambda i: (i, 0)),
        ],
        out_shape=[
            jax.ShapeDtypeStruct((_N, _D), jnp.float32),
            jax.ShapeDtypeStruct((_N, 3), jnp.float32),
        ],
    )(hn, *maccs, x, v_out, wn1b, bn1, wn2, bn2)


def kernel(h, x, edge_index, edge_attr, v_init, We1, be1, We2, be2, Wc1, bc1,
           Wc2, Wn1, bn1, Wn2, bn2, Wv1, bv1, Wv2, Wi1, bi1, Wi2, bi2):
    f32 = jnp.float32
    row = edge_index[0]
    col = edge_index[1]

    t1, t2, hn, v_out = _tc_pre(
        h, x, v_init,
        We1[:_D], be1.reshape(1, _H), We1[_D:2 * _D], Wn1[:_D],
        Wv1, bv1.reshape(1, _H), Wv2[:, 0].reshape(1, _H))

    k17 = jnp.concatenate(
        [jnp.tile(We1[2 * _D].reshape(1, _H), (3, 1)),
         jnp.zeros((13, _H), f32),
         We1[2 * _D + 1].reshape(1, _H)], axis=0)
    sel3 = jnp.zeros((1, 16), f32).at[0, 3].set(1.0)
    zm = jnp.zeros((_N, _TD), f32)

    maccs = []
    for s in range(_P):
        gx = _sc_gather(row, col, t1, t2, s * _ES)
        mx = _tc_edge(
            gx, edge_attr,
            k17, sel3,
            We2, be2.reshape(1, _H),
            Wi1, bi1.reshape(1, _H // 2), Wi2,
            bi2.reshape(1, 1),
            Wc1, bc1.reshape(1, _H), Wc2,
            s * (_ES // _BE))
        maccs.append(_sc_scatter(row, mx, zm, s * _ES))

    h_out, x_out = _tc_final(
        hn, maccs, x, v_out,
        Wn1[_D:], bn1.reshape(1, _H), Wn2, bn2.reshape(1, _D))

    return (h_out, x_out, v_out)
